# baseline (speedup 1.0000x reference)
import functools

import jax
import jax.numpy as jnp
from jax import lax
from jax.experimental import pallas as pl
from jax.experimental.pallas import tpu as pltpu
from jax.experimental.pallas import tpu_sc as plsc

_LAM = 0.75
_GAMMA = 1.5
_NEG_WEIGHT = 300.0
_L2_REG_WEIGHT = 1e-4

_NC = 2
_NS = 16
_NW = _NC * _NS
_CH = 128


def _sc_gather(emb_table, beta_uD, beta_iD, nbr_flat, sc_flat,
               src, pos, neg_t, ii_flat_idx):
    b = src.shape[0]
    nneg_total = neg_t.shape[0]
    nii_total = ii_flat_idx.shape[0]
    d = emb_table.shape[1]

    b_w = b // _NW
    neg_w = nneg_total // _NW
    ii_w = nii_total // _NW
    assert b_w == _CH and neg_w % _CH == 0 and ii_w % _CH == 0

    mesh = plsc.VectorSubcoreMesh(core_axis_name="c", subcore_axis_name="s")
    f32 = jnp.float32
    DMA = pltpu.SemaphoreType.DMA

    @functools.partial(
        pl.kernel,
        out_type=[
            jax.ShapeDtypeStruct((b, d), f32),
            jax.ShapeDtypeStruct((b, d), f32),
            jax.ShapeDtypeStruct((nneg_total, d), f32),
            jax.ShapeDtypeStruct((nii_total, d), f32),
            jax.ShapeDtypeStruct((nii_total,), f32),
            jax.ShapeDtypeStruct((b,), f32),
            jax.ShapeDtypeStruct((b,), f32),
            jax.ShapeDtypeStruct((nneg_total,), f32),
        ],
        mesh=mesh,
        scratch_types=[
            pltpu.VMEM((_CH,), jnp.int32),
            pltpu.VMEM((_CH,), jnp.int32),
            pltpu.VMEM((_CH, 128), f32),
            pltpu.VMEM((_CH, 128), f32),
            pltpu.VMEM((_CH,), f32),
            pltpu.VMEM((_CH,), f32),
            pltpu.VMEM((ii_w,), jnp.int32),
            DMA, DMA,
            DMA, DMA,
            DMA, DMA,
            DMA, DMA,
        ],
    )
    def gather_kernel(emb_h, bu_h, bi_h, nbr_h, scf_h, src_h, pos_h, negt_h,
                      iidx_h, osrc_h, opos_h, oneg_h, oii_h, oiisc_h,
                      obu_h, obip_h, obin_h,
                      idx0, idx1, rows0, rows1, val0, val1, nbr_v,
                      g0, g1, v0, v1, w0, w1, x0, x1):
        wid = lax.axis_index("s") * _NC + lax.axis_index("c")
        idx = (idx0, idx1)
        rows = (rows0, rows1)
        val = (val0, val1)
        gsem = (g0, g1)
        vsem = (v0, v1)
        wsem = (w0, w1)
        xsem = (x0, x1)

        def run_job(nchunks, start, finish):
            if nchunks == 2:
                start(0, 0, False)
                start(1, 1, False)
                finish(0, 0)
                finish(1, 1)
            else:
                start(0, 0, False)
                start(1, 1, False)

                @pl.loop(0, nchunks // 2)
                def _(g):
                    for s in range(2):
                        c = 2 * g + s
                        finish(s, c)

                        @pl.when(c + 2 < nchunks)
                        def _():
                            start(s, c + 2, True)

        def drain(sems_and_waits):
            for sem, src_ref, dst_ref in sems_and_waits:
                pltpu.make_async_copy(src_ref, dst_ref, sem).wait()

        abase = wid * b_w

        def a_start(s, c, wait_reuse):
            idx_h = src_h if c == 0 else pos_h
            beta_h = bu_h if c == 0 else bi_h
            pltpu.sync_copy(idx_h.at[pl.ds(abase, _CH)], idx[s])
            pltpu.make_async_copy(emb_h.at[idx[s]], rows[s], gsem[s]).start()
            pltpu.make_async_copy(beta_h.at[idx[s]], val[s], vsem[s]).start()

        def a_finish(s, c):
            orow_h = osrc_h if c == 0 else opos_h
            oval_h = obu_h if c == 0 else obip_h
            beta_h = bu_h if c == 0 else bi_h
            pltpu.make_async_copy(emb_h.at[idx[s]], rows[s], gsem[s]).wait()
            pltpu.make_async_copy(rows[s], orow_h.at[pl.ds(abase, _CH)],
                                  wsem[s]).start()
            pltpu.make_async_copy(beta_h.at[idx[s]], val[s], vsem[s]).wait()
            pltpu.make_async_copy(val[s], oval_h.at[pl.ds(abase, _CH)],
                                  xsem[s]).start()

        run_job(2, a_start, a_finish)
        drain([(wsem[0], rows[0], osrc_h.at[pl.ds(abase, _CH)]),
               (wsem[1], rows[1], opos_h.at[pl.ds(abase, _CH)]),
               (xsem[0], val[0], obu_h.at[pl.ds(abase, _CH)]),
               (xsem[1], val[1], obip_h.at[pl.ds(abase, _CH)])])

        def b_start(s, c, wait_reuse):
            nbase = wid * neg_w + c * _CH
            if wait_reuse:
                pltpu.make_async_copy(rows[s], oneg_h.at[pl.ds(0, _CH)],
                                      wsem[s]).wait()
                pltpu.make_async_copy(val[s], obin_h.at[pl.ds(0, _CH)],
                                      xsem[s]).wait()
            pltpu.sync_copy(negt_h.at[pl.ds(nbase, _CH)], idx[s])
            pltpu.make_async_copy(emb_h.at[idx[s]], rows[s], gsem[s]).start()
            pltpu.make_async_copy(bi_h.at[idx[s]], val[s], vsem[s]).start()

        def b_finish(s, c):
            nbase = wid * neg_w + c * _CH
            pltpu.make_async_copy(emb_h.at[idx[s]], rows[s], gsem[s]).wait()
            pltpu.make_async_copy(rows[s], oneg_h.at[pl.ds(nbase, _CH)],
                                  wsem[s]).start()
            pltpu.make_async_copy(bi_h.at[idx[s]], val[s], vsem[s]).wait()
            pltpu.make_async_copy(val[s], obin_h.at[pl.ds(nbase, _CH)],
                                  xsem[s]).start()

        run_job(neg_w // _CH, b_start, b_finish)
        drain([(wsem[0], rows[0], oneg_h.at[pl.ds(0, _CH)]),
               (wsem[1], rows[1], oneg_h.at[pl.ds(0, _CH)]),
               (xsem[0], val[0], obin_h.at[pl.ds(0, _CH)]),
               (xsem[1], val[1], obin_h.at[pl.ds(0, _CH)])])

        def c_start(s, c, wait_reuse):
            ibase = wid * ii_w + c * _CH
            if wait_reuse:
                pltpu.make_async_copy(val[s], oiisc_h.at[pl.ds(0, _CH)],
                                      xsem[s]).wait()
            pltpu.sync_copy(iidx_h.at[pl.ds(ibase, _CH)], idx[s])
            pltpu.make_async_copy(nbr_h.at[idx[s]],
                                  nbr_v.at[pl.ds(c * _CH, _CH)],
                                  gsem[s]).start()
            pltpu.make_async_copy(scf_h.at[idx[s]], val[s], vsem[s]).start()

        def c_finish(s, c):
            ibase = wid * ii_w + c * _CH
            pltpu.make_async_copy(nbr_h.at[idx[s]],
                                  nbr_v.at[pl.ds(c * _CH, _CH)],
                                  gsem[s]).wait()
            pltpu.make_async_copy(scf_h.at[idx[s]], val[s], vsem[s]).wait()
            pltpu.make_async_copy(val[s], oiisc_h.at[pl.ds(ibase, _CH)],
                                  xsem[s]).start()

        run_job(ii_w // _CH, c_start, c_finish)
        drain([(xsem[0], val[0], oiisc_h.at[pl.ds(0, _CH)]),
               (xsem[1], val[1], oiisc_h.at[pl.ds(0, _CH)])])

        def d_start(s, c, wait_reuse):
            if wait_reuse:
                pltpu.make_async_copy(rows[s], oii_h.at[pl.ds(0, _CH)],
                                      wsem[s]).wait()
            pltpu.make_async_copy(emb_h.at[nbr_v.at[pl.ds(c * _CH, _CH)]],
                                  rows[s], gsem[s]).start()

        def d_finish(s, c):
            ibase = wid * ii_w + c * _CH
            pltpu.make_async_copy(emb_h.at[nbr_v.at[pl.ds(c * _CH, _CH)]],
                                  rows[s], gsem[s]).wait()
            pltpu.make_async_copy(rows[s], oii_h.at[pl.ds(ibase, _CH)],
                                  wsem[s]).start()

        run_job(ii_w // _CH, d_start, d_finish)
        drain([(wsem[0], rows[0], oii_h.at[pl.ds(0, _CH)]),
               (wsem[1], rows[1], oii_h.at[pl.ds(0, _CH)])])

    return gather_kernel(emb_table, beta_uD, beta_iD, nbr_flat, sc_flat,
                         src, pos, neg_t, ii_flat_idx)


def _loss_block(src_ref, pos_ref, neg_ref, ii_ref, bu_ref, bip_ref,
                bin_ref, iisc_ref, bdn_ref, bdi_ref, one_ref, out_ref):
    b = pl.program_id(0)
    bb = src_ref.shape[1]
    d = src_ref.shape[2]
    nneg = neg_ref.shape[2] // d
    topk = ii_ref.shape[2] // d

    src = src_ref[0]
    pos = pos_ref[0]
    neg = neg_ref[0]
    ii = ii_ref[0]
    bdn = bdn_ref[...]
    bdi = bdi_ref[...]
    one = one_ref[...]

    src_n = jnp.concatenate([src] * nneg, axis=1)
    src_k = jnp.concatenate([src] * topk, axis=1)
    p_neg = src_n * neg
    p_ii = src_k * ii
    p_pos = src * pos
    neg_score = jnp.dot(p_neg, bdn)
    ii_score = jnp.dot(p_ii, bdi)
    pos_score = jnp.dot(p_pos, one)

    sq_neg = neg * neg
    sq_ii = ii * ii
    sq_sp = src * src + pos * pos
    l2n = jnp.dot(sq_neg, bdn)
    l2i = jnp.dot(sq_ii, bdi)
    l2sp = jnp.dot(sq_sp, one)

    bu = bu_ref[0]
    bip = bip_ref[0]
    bin_ = bin_ref[0]
    iisc = iisc_ref[0]

    def bce(x, target):
        return (jnp.maximum(x, 0.0) - x * target
                + jnp.log1p(jnp.exp(-jnp.abs(x))))

    pos_coe = 1.0 + _LAM * bu * bip
    neg_coe = 1.0 + _LAM * bu * bin_
    s_pos = jnp.sum(bce(pos_score, 1.0) * pos_coe)
    s_neg = jnp.sum(bce(neg_score, 0.0) * neg_coe)

    log_sig = jnp.minimum(ii_score, 0.0) - jnp.log1p(jnp.exp(-jnp.abs(ii_score)))
    s_i = jnp.sum(iisc * log_sig)

    s_l2 = jnp.sum(l2n) + jnp.sum(l2i) + jnp.sum(l2sp)

    n_pos = bb * pl.num_programs(0)
    n_neg = nneg * n_pos
    contrib = (s_pos / n_pos
               + (_NEG_WEIGHT / n_neg) * s_neg
               - _GAMMA * s_i
               + (0.5 * _L2_REG_WEIGHT) * s_l2)

    @pl.when(b == 0)
    def _():
        out_ref[0, 0] = 0.0
    out_ref[0, 0] += contrib


def _tc_loss(src_rows, pos_rows, neg_rows, ii_rows, ii_sc, bu, bip, bin_):
    b, d = src_rows.shape
    nneg = neg_rows.shape[0] // b
    topk = ii_rows.shape[0] // b
    bb = 128
    nblk = b // bb

    src3 = src_rows.reshape(nblk, bb, d)
    pos3 = pos_rows.reshape(nblk, bb, d)
    neg4 = neg_rows.reshape(nblk, bb, nneg * d)
    ii4 = ii_rows.reshape(nblk, bb, topk * d)
    bu3 = bu.reshape(nblk, bb, 1)
    bip3 = bip.reshape(nblk, bb, 1)
    bin3 = bin_.reshape(nblk, bb, nneg)
    iisc3 = ii_sc.reshape(nblk, bb, topk)

    def _bd(n):
        return (jax.lax.broadcasted_iota(jnp.int32, (n * d, n), 0) // d
                == jax.lax.broadcasted_iota(jnp.int32, (n * d, n), 1)
                ).astype(jnp.float32)

    bdn = _bd(nneg)
    bdi = _bd(topk)
    one = jnp.ones((d, 1), jnp.float32)

    out = pl.pallas_call(
        _loss_block,
        grid=(nblk,),
        in_specs=[
            pl.BlockSpec((1, bb, d), lambda i: (i, 0, 0)),
            pl.BlockSpec((1, bb, d), lambda i: (i, 0, 0)),
            pl.BlockSpec((1, bb, nneg * d), lambda i: (i, 0, 0)),
            pl.BlockSpec((1, bb, topk * d), lambda i: (i, 0, 0)),
            pl.BlockSpec((1, bb, 1), lambda i: (i, 0, 0)),
            pl.BlockSpec((1, bb, 1), lambda i: (i, 0, 0)),
            pl.BlockSpec((1, bb, nneg), lambda i: (i, 0, 0)),
            pl.BlockSpec((1, bb, topk), lambda i: (i, 0, 0)),
            pl.BlockSpec((nneg * d, nneg), lambda i: (0, 0)),
            pl.BlockSpec((topk * d, topk), lambda i: (0, 0)),
            pl.BlockSpec((d, 1), lambda i: (0, 0)),
        ],
        out_specs=pl.BlockSpec((1, 1), lambda i: (0, 0),
                               memory_space=pltpu.SMEM),
        out_shape=jax.ShapeDtypeStruct((1, 1), jnp.float32),
    )(src3, pos3, neg4, ii4, bu3, bip3, bin3, iisc3, bdn, bdi, one)
    return out[0, 0]


def kernel(emb_table, beta_uD, beta_iD, ii_topk_similarity_scores,
           src, pos, neg, ii_topk_neighbors):
    b, nneg = neg.shape
    topk = ii_topk_neighbors.shape[1]

    neg_t = neg.reshape(-1).astype(jnp.int32)
    ii_flat_idx = (pos[:, None] * topk
                   + jnp.arange(topk, dtype=pos.dtype)[None, :]
                   ).reshape(-1).astype(jnp.int32)
    nbr_flat = ii_topk_neighbors.reshape(-1).astype(jnp.int32)
    sc_flat = ii_topk_similarity_scores.reshape(-1)

    (src_rows, pos_rows, neg_rows, ii_rows, ii_sc, bu, bip, bin_) = _sc_gather(
        emb_table, beta_uD, beta_iD, nbr_flat, sc_flat,
        src.astype(jnp.int32), pos.astype(jnp.int32), neg_t, ii_flat_idx)

    return _tc_loss(src_rows, pos_rows, neg_rows, ii_rows, ii_sc, bu, bip, bin_)


# speedup vs baseline: 1.2613x; 1.2613x over previous
import dataclasses
import functools

import jax
import jax.numpy as jnp
from jax import lax
from jax.experimental import pallas as pl
from jax.experimental.pallas import tpu as pltpu
from jax.experimental.pallas import tpu_sc as plsc

_LAM = 0.75
_GAMMA = 1.5
_NEG_WEIGHT = 300.0
_L2_REG_WEIGHT = 1e-4

_NC = 2
_NS = 16
_NW = _NC * _NS
_CH = 128


def _sc_gather(emb_table, beta_uD, beta_iD, cat_pad, src, pos, neg_t, topk):
    b = src.shape[0]
    nneg_total = neg_t.shape[0]
    nii_total = b * topk
    d = emb_table.shape[1]

    b_w = b // _NW
    neg_w = nneg_total // _NW
    ii_w = nii_total // _NW
    assert b_w == _CH and neg_w % _CH == 0 and ii_w % _CH == 0

    mesh = plsc.VectorSubcoreMesh(core_axis_name="c", subcore_axis_name="s")
    f32 = jnp.float32
    DMA = pltpu.SemaphoreType.DMA
    cp = pltpu.CompilerParams()
    if "needs_layout_passes" in pltpu.CompilerParams.__dataclass_fields__:
        cp = dataclasses.replace(cp, needs_layout_passes=False)

    @functools.partial(
        pl.kernel,
        compiler_params=cp,
        out_type=[
            jax.ShapeDtypeStruct((b, d), f32),
            jax.ShapeDtypeStruct((b, d), f32),
            jax.ShapeDtypeStruct((nneg_total, d), f32),
            jax.ShapeDtypeStruct((nii_total, d), f32),
            jax.ShapeDtypeStruct((b, d), jnp.int32),
            jax.ShapeDtypeStruct((b,), f32),
            jax.ShapeDtypeStruct((b,), f32),
            jax.ShapeDtypeStruct((nneg_total,), f32),
        ],
        mesh=mesh,
        scratch_types=[
            pltpu.VMEM((_CH,), jnp.int32),
            pltpu.VMEM((_CH,), jnp.int32),
            pltpu.VMEM((_CH, 128), f32),
            pltpu.VMEM((_CH, 128), f32),
            pltpu.VMEM((_CH,), f32),
            pltpu.VMEM((_CH,), f32),
            pltpu.VMEM((ii_w,), jnp.int32),
            pltpu.VMEM((_CH, 128), jnp.int32),
            DMA, DMA,
            DMA, DMA,
            DMA, DMA,
            DMA, DMA,
        ],
    )
    def gather_kernel(emb_h, bu_h, bi_h, cat_h, src_h, pos_h, negt_h,
                      osrc_h, opos_h, oneg_h, oii_h, ocat_h,
                      obu_h, obip_h, obin_h,
                      idx0, idx1, rows0, rows1, val0, val1, nbr_v, cat_v,
                      g0, g1, v0, v1, w0, w1, x0, x1):
        wid = lax.axis_index("s") * _NC + lax.axis_index("c")
        idx = (idx0, idx1)
        rows = (rows0, rows1)
        val = (val0, val1)
        gsem = (g0, g1)
        vsem = (v0, v1)
        wsem = (w0, w1)
        xsem = (x0, x1)

        def run_job(nchunks, start, finish):
            if nchunks == 2:
                start(0, 0, False)
                start(1, 1, False)
                finish(0, 0)
                finish(1, 1)
            else:
                start(0, 0, False)
                start(1, 1, False)

                @pl.loop(0, nchunks // 2)
                def _(g):
                    for s in range(2):
                        c = 2 * g + s
                        finish(s, c)

                        @pl.when(c + 2 < nchunks)
                        def _():
                            start(s, c + 2, True)

        def drain(sems_and_waits):
            for sem, src_ref, dst_ref in sems_and_waits:
                pltpu.make_async_copy(src_ref, dst_ref, sem).wait()

        abase = wid * b_w

        def a_start(s, c, wait_reuse):
            idx_h = src_h if c == 0 else pos_h
            beta_h = bu_h if c == 0 else bi_h
            pltpu.sync_copy(idx_h.at[pl.ds(abase, _CH)], idx[s])
            pltpu.make_async_copy(emb_h.at[idx[s]], rows[s], gsem[s]).start()
            pltpu.make_async_copy(beta_h.at[idx[s]], val[s], vsem[s]).start()

        def a_finish(s, c):
            orow_h = osrc_h if c == 0 else opos_h
            oval_h = obu_h if c == 0 else obip_h
            beta_h = bu_h if c == 0 else bi_h
            pltpu.make_async_copy(emb_h.at[idx[s]], rows[s], gsem[s]).wait()
            pltpu.make_async_copy(rows[s], orow_h.at[pl.ds(abase, _CH)],
                                  wsem[s]).start()
            pltpu.make_async_copy(beta_h.at[idx[s]], val[s], vsem[s]).wait()
            pltpu.make_async_copy(val[s], oval_h.at[pl.ds(abase, _CH)],
                                  xsem[s]).start()

        run_job(2, a_start, a_finish)
        drain([(wsem[0], rows[0], osrc_h.at[pl.ds(abase, _CH)]),
               (wsem[1], rows[1], opos_h.at[pl.ds(abase, _CH)]),
               (xsem[0], val[0], obu_h.at[pl.ds(abase, _CH)]),
               (xsem[1], val[1], obip_h.at[pl.ds(abase, _CH)])])

        def b_start(s, c, wait_reuse):
            nbase = wid * neg_w + c * _CH
            if wait_reuse:
                pltpu.make_async_copy(rows[s], oneg_h.at[pl.ds(0, _CH)],
                                      wsem[s]).wait()
                pltpu.make_async_copy(val[s], obin_h.at[pl.ds(0, _CH)],
                                      xsem[s]).wait()
            pltpu.sync_copy(negt_h.at[pl.ds(nbase, _CH)], idx[s])
            pltpu.make_async_copy(emb_h.at[idx[s]], rows[s], gsem[s]).start()
            pltpu.make_async_copy(bi_h.at[idx[s]], val[s], vsem[s]).start()

        def b_finish(s, c):
            nbase = wid * neg_w + c * _CH
            pltpu.make_async_copy(emb_h.at[idx[s]], rows[s], gsem[s]).wait()
            pltpu.make_async_copy(rows[s], oneg_h.at[pl.ds(nbase, _CH)],
                                  wsem[s]).start()
            pltpu.make_async_copy(bi_h.at[idx[s]], val[s], vsem[s]).wait()
            pltpu.make_async_copy(val[s], obin_h.at[pl.ds(nbase, _CH)],
                                  xsem[s]).start()

        run_job(neg_w // _CH, b_start, b_finish)
        drain([(wsem[0], rows[0], oneg_h.at[pl.ds(0, _CH)]),
               (wsem[1], rows[1], oneg_h.at[pl.ds(0, _CH)]),
               (xsem[0], val[0], obin_h.at[pl.ds(0, _CH)]),
               (xsem[1], val[1], obin_h.at[pl.ds(0, _CH)])])

        pltpu.sync_copy(pos_h.at[pl.ds(abase, _CH)], idx0)
        pltpu.sync_copy(cat_h.at[idx0], cat_v)
        pltpu.make_async_copy(cat_v, ocat_h.at[pl.ds(abase, _CH)],
                              xsem[0]).start()
        it16 = lax.iota(jnp.int32, 16)

        @pl.loop(0, ii_w // 16)
        def _ext(t):
            jj = t * 16 + it16
            row = jj // topk
            lane = jj % topk
            nbr_v[pl.ds(t * 16, 16)] = plsc.load_gather(cat_v, [row, lane])

        pltpu.make_async_copy(cat_v, ocat_h.at[pl.ds(abase, _CH)],
                              xsem[0]).wait()

        def d_start(s, c, wait_reuse):
            if wait_reuse:
                pltpu.make_async_copy(rows[s], oii_h.at[pl.ds(0, _CH)],
                                      wsem[s]).wait()
            pltpu.make_async_copy(emb_h.at[nbr_v.at[pl.ds(c * _CH, _CH)]],
                                  rows[s], gsem[s]).start()

        def d_finish(s, c):
            ibase = wid * ii_w + c * _CH
            pltpu.make_async_copy(emb_h.at[nbr_v.at[pl.ds(c * _CH, _CH)]],
                                  rows[s], gsem[s]).wait()
            pltpu.make_async_copy(rows[s], oii_h.at[pl.ds(ibase, _CH)],
                                  wsem[s]).start()

        run_job(ii_w // _CH, d_start, d_finish)
        drain([(wsem[0], rows[0], oii_h.at[pl.ds(0, _CH)]),
               (wsem[1], rows[1], oii_h.at[pl.ds(0, _CH)])])

    return gather_kernel(emb_table, beta_uD, beta_iD, cat_pad,
                         src, pos, neg_t)


def _loss_block(src_ref, pos_ref, neg_ref, ii_ref, bu_ref, bip_ref,
                bin_ref, ocat_ref, bdn_ref, bdi_ref, one_ref, out_ref):
    b = pl.program_id(0)
    bb = src_ref.shape[1]
    d = src_ref.shape[2]
    nneg = neg_ref.shape[2] // d
    topk = ii_ref.shape[2] // d

    src = src_ref[0]
    pos = pos_ref[0]
    neg = neg_ref[0]
    ii = ii_ref[0]
    bdn = bdn_ref[...]
    bdi = bdi_ref[...]
    one = one_ref[...]

    src_n = jnp.concatenate([src] * nneg, axis=1)
    src_k = jnp.concatenate([src] * topk, axis=1)
    p_neg = src_n * neg
    p_ii = src_k * ii
    p_pos = src * pos
    neg_score = jnp.dot(p_neg, bdn)
    ii_score = jnp.dot(p_ii, bdi)
    pos_score = jnp.dot(p_pos, one)

    sq_neg = neg * neg
    sq_ii = ii * ii
    sq_sp = src * src + pos * pos
    l2n = jnp.dot(sq_neg, bdn)
    l2i = jnp.dot(sq_ii, bdi)
    l2sp = jnp.dot(sq_sp, one)

    bu = bu_ref[pl.ds(b, 1), :].reshape(bb, 1)
    bip = bip_ref[pl.ds(b, 1), :].reshape(bb, 1)
    bin_ = bin_ref[0]
    iisc = jax.lax.bitcast_convert_type(
        ocat_ref[0][:, topk:2 * topk], jnp.float32)

    def bce(x, target):
        return (jnp.maximum(x, 0.0) - x * target
                + jnp.log1p(jnp.exp(-jnp.abs(x))))

    pos_coe = 1.0 + _LAM * bu * bip
    neg_coe = 1.0 + _LAM * bu * bin_
    s_pos = jnp.sum(bce(pos_score, 1.0) * pos_coe)
    s_neg = jnp.sum(bce(neg_score, 0.0) * neg_coe)

    log_sig = jnp.minimum(ii_score, 0.0) - jnp.log1p(jnp.exp(-jnp.abs(ii_score)))
    s_i = jnp.sum(iisc * log_sig)

    s_l2 = jnp.sum(l2n) + jnp.sum(l2i) + jnp.sum(l2sp)

    n_pos = bb * pl.num_programs(0)
    n_neg = nneg * n_pos
    contrib = (s_pos / n_pos
               + (_NEG_WEIGHT / n_neg) * s_neg
               - _GAMMA * s_i
               + (0.5 * _L2_REG_WEIGHT) * s_l2)

    @pl.when(b == 0)
    def _():
        out_ref[0, 0] = 0.0
    out_ref[0, 0] += contrib


def _tc_loss(src_rows, pos_rows, neg_rows, ii_rows, ocat, bu, bip, bin_,
             topk):
    b, d = src_rows.shape
    nneg = neg_rows.shape[0] // b
    bb = 128
    nblk = b // bb

    src3 = src_rows.reshape(nblk, bb, d)
    pos3 = pos_rows.reshape(nblk, bb, d)
    neg4 = neg_rows.reshape(nblk, bb, nneg * d)
    ii4 = ii_rows.reshape(nblk, bb, topk * d)
    ocat3 = ocat.reshape(nblk, bb, d)
    bu2 = bu.reshape(nblk, bb)
    bip2 = bip.reshape(nblk, bb)
    bin2 = bin_.reshape(nblk, bb, nneg)

    def _bd(n):
        return (jax.lax.broadcasted_iota(jnp.int32, (n * d, n), 0) // d
                == jax.lax.broadcasted_iota(jnp.int32, (n * d, n), 1)
                ).astype(jnp.float32)

    bdn = _bd(nneg)
    bdi = _bd(topk)
    one = jnp.ones((d, 1), jnp.float32)

    out = pl.pallas_call(
        _loss_block,
        grid=(nblk,),
        in_specs=[
            pl.BlockSpec((1, bb, d), lambda i: (i, 0, 0)),
            pl.BlockSpec((1, bb, d), lambda i: (i, 0, 0)),
            pl.BlockSpec((1, bb, nneg * d), lambda i: (i, 0, 0)),
            pl.BlockSpec((1, bb, topk * d), lambda i: (i, 0, 0)),
            pl.BlockSpec((nblk, bb), lambda i: (0, 0)),
            pl.BlockSpec((nblk, bb), lambda i: (0, 0)),
            pl.BlockSpec((1, bb, nneg), lambda i: (i, 0, 0)),
            pl.BlockSpec((1, bb, d), lambda i: (i, 0, 0)),
            pl.BlockSpec((nneg * d, nneg), lambda i: (0, 0)),
            pl.BlockSpec((topk * d, topk), lambda i: (0, 0)),
            pl.BlockSpec((d, 1), lambda i: (0, 0)),
        ],
        out_specs=pl.BlockSpec((1, 1), lambda i: (0, 0),
                               memory_space=pltpu.SMEM),
        out_shape=jax.ShapeDtypeStruct((1, 1), jnp.float32),
    )(src3, pos3, neg4, ii4, bu2, bip2, bin2, ocat3, bdn, bdi, one)
    return out[0, 0]


def kernel(emb_table, beta_uD, beta_iD, ii_topk_similarity_scores,
           src, pos, neg, ii_topk_neighbors):
    b, nneg = neg.shape
    topk = ii_topk_neighbors.shape[1]

    d = emb_table.shape[1]

    neg_t = neg.reshape(-1).astype(jnp.int32)

    cat_pad = jnp.pad(
        jnp.concatenate(
            [ii_topk_neighbors.astype(jnp.int32),
             jax.lax.bitcast_convert_type(ii_topk_similarity_scores,
                                          jnp.int32)], axis=1),
        ((0, 0), (0, d - 2 * topk)))

    (src_rows, pos_rows, neg_rows, ii_rows, ocat, bu, bip, bin_) = _sc_gather(
        emb_table, beta_uD, beta_iD, cat_pad,
        src.astype(jnp.int32), pos.astype(jnp.int32), neg_t, topk)

    return _tc_loss(src_rows, pos_rows, neg_rows, ii_rows, ocat, bu, bip,
                    bin_, topk)


# speedup vs baseline: 1.2716x; 1.0081x over previous
import dataclasses
import functools

import jax
import jax.numpy as jnp
from jax import lax
from jax.experimental import pallas as pl
from jax.experimental.pallas import tpu as pltpu
from jax.experimental.pallas import tpu_sc as plsc

_LAM = 0.75
_GAMMA = 1.5
_NEG_WEIGHT = 300.0
_L2_REG_WEIGHT = 1e-4

_NC = 2
_NS = 16
_NW = _NC * _NS
_CH = 128


def _sc_gather(emb_table, beta_uD, beta_iD, cat_pad, src, pos, neg_t, topk):
    b = src.shape[0]
    nneg_total = neg_t.shape[0]
    nii_total = b * topk
    d = emb_table.shape[1]

    b_w = b // _NW
    neg_w = nneg_total // _NW
    ii_w = nii_total // _NW
    ch = _CH if neg_w % _CH == 0 else 80
    assert b_w <= _CH and neg_w % ch == 0 and ii_w % ch == 0
    assert (neg_w // ch) % 2 == 0 and (ii_w // ch) % 2 == 0

    mesh = plsc.VectorSubcoreMesh(core_axis_name="c", subcore_axis_name="s")
    f32 = jnp.float32
    DMA = pltpu.SemaphoreType.DMA
    cp = pltpu.CompilerParams()
    if "needs_layout_passes" in pltpu.CompilerParams.__dataclass_fields__:
        cp = dataclasses.replace(cp, needs_layout_passes=False)

    @functools.partial(
        pl.kernel,
        compiler_params=cp,
        out_type=[
            jax.ShapeDtypeStruct((b, d), f32),
            jax.ShapeDtypeStruct((b, d), f32),
            jax.ShapeDtypeStruct((nneg_total, d), f32),
            jax.ShapeDtypeStruct((nii_total, d), f32),
            jax.ShapeDtypeStruct((b, d), jnp.int32),
            jax.ShapeDtypeStruct((b,), f32),
            jax.ShapeDtypeStruct((b,), f32),
            jax.ShapeDtypeStruct((nneg_total,), f32),
        ],
        mesh=mesh,
        scratch_types=[
            pltpu.VMEM((_CH,), jnp.int32),
            pltpu.VMEM((_CH,), jnp.int32),
            pltpu.VMEM((_CH, 128), f32),
            pltpu.VMEM((_CH, 128), f32),
            pltpu.VMEM((_CH,), f32),
            pltpu.VMEM((_CH,), f32),
            pltpu.VMEM((ii_w,), jnp.int32),
            pltpu.VMEM((_CH, 128), jnp.int32),
            DMA, DMA,
            DMA, DMA,
            DMA, DMA,
            DMA, DMA,
        ],
    )
    def gather_kernel(emb_h, bu_h, bi_h, cat_h, src_h, pos_h, negt_h,
                      osrc_h, opos_h, oneg_h, oii_h, ocat_h,
                      obu_h, obip_h, obin_h,
                      idx0, idx1, rows0, rows1, val0, val1, nbr_v, cat_v,
                      g0, g1, v0, v1, w0, w1, x0, x1):
        wid = lax.axis_index("s") * _NC + lax.axis_index("c")
        idx = (idx0, idx1)
        rows = (rows0, rows1)
        val = (val0, val1)
        gsem = (g0, g1)
        vsem = (v0, v1)
        wsem = (w0, w1)
        xsem = (x0, x1)

        def run_job(nchunks, start, finish):
            if nchunks == 2:
                start(0, 0, False)
                start(1, 1, False)
                finish(0, 0)
                finish(1, 1)
            else:
                start(0, 0, False)
                start(1, 1, False)

                @pl.loop(0, nchunks // 2)
                def _(g):
                    for s in range(2):
                        c = 2 * g + s
                        finish(s, c)

                        @pl.when(c + 2 < nchunks)
                        def _():
                            start(s, c + 2, True)

        def drain(sems_and_waits):
            for sem, src_ref, dst_ref in sems_and_waits:
                pltpu.make_async_copy(src_ref, dst_ref, sem).wait()

        abase = wid * b_w

        idxw = [r.at[pl.ds(0, b_w)] for r in idx]
        rowsw = [r.at[pl.ds(0, b_w)] for r in rows]
        valw = [r.at[pl.ds(0, b_w)] for r in val]

        def a_start(s, c, wait_reuse):
            idx_h = src_h if c == 0 else pos_h
            beta_h = bu_h if c == 0 else bi_h
            pltpu.sync_copy(idx_h.at[pl.ds(abase, b_w)], idxw[s])
            pltpu.make_async_copy(emb_h.at[idxw[s]], rowsw[s], gsem[s]).start()
            pltpu.make_async_copy(beta_h.at[idxw[s]], valw[s], vsem[s]).start()

        def a_finish(s, c):
            orow_h = osrc_h if c == 0 else opos_h
            oval_h = obu_h if c == 0 else obip_h
            beta_h = bu_h if c == 0 else bi_h
            pltpu.make_async_copy(emb_h.at[idxw[s]], rowsw[s], gsem[s]).wait()
            pltpu.make_async_copy(rowsw[s], orow_h.at[pl.ds(abase, b_w)],
                                  wsem[s]).start()
            pltpu.make_async_copy(beta_h.at[idxw[s]], valw[s], vsem[s]).wait()
            pltpu.make_async_copy(valw[s], oval_h.at[pl.ds(abase, b_w)],
                                  xsem[s]).start()

        run_job(2, a_start, a_finish)
        drain([(wsem[0], rowsw[0], osrc_h.at[pl.ds(abase, b_w)]),
               (wsem[1], rowsw[1], opos_h.at[pl.ds(abase, b_w)]),
               (xsem[0], valw[0], obu_h.at[pl.ds(abase, b_w)]),
               (xsem[1], valw[1], obip_h.at[pl.ds(abase, b_w)])])

        idxc = [r.at[pl.ds(0, ch)] for r in idx]
        rowsc = [r.at[pl.ds(0, ch)] for r in rows]
        valc = [r.at[pl.ds(0, ch)] for r in val]

        def b_start(s, c, wait_reuse):
            nbase = wid * neg_w + c * ch
            if wait_reuse:
                pltpu.make_async_copy(rowsc[s], oneg_h.at[pl.ds(0, ch)],
                                      wsem[s]).wait()
                pltpu.make_async_copy(valc[s], obin_h.at[pl.ds(0, ch)],
                                      xsem[s]).wait()
            pltpu.sync_copy(negt_h.at[pl.ds(nbase, ch)], idxc[s])
            pltpu.make_async_copy(emb_h.at[idxc[s]], rowsc[s], gsem[s]).start()
            pltpu.make_async_copy(bi_h.at[idxc[s]], valc[s], vsem[s]).start()

        def b_finish(s, c):
            nbase = wid * neg_w + c * ch
            pltpu.make_async_copy(emb_h.at[idxc[s]], rowsc[s], gsem[s]).wait()
            pltpu.make_async_copy(rowsc[s], oneg_h.at[pl.ds(nbase, ch)],
                                  wsem[s]).start()
            pltpu.make_async_copy(bi_h.at[idxc[s]], valc[s], vsem[s]).wait()
            pltpu.make_async_copy(valc[s], obin_h.at[pl.ds(nbase, ch)],
                                  xsem[s]).start()

        run_job(neg_w // ch, b_start, b_finish)
        drain([(wsem[0], rowsc[0], oneg_h.at[pl.ds(0, ch)]),
               (wsem[1], rowsc[1], oneg_h.at[pl.ds(0, ch)]),
               (xsem[0], valc[0], obin_h.at[pl.ds(0, ch)]),
               (xsem[1], valc[1], obin_h.at[pl.ds(0, ch)])])

        catw = cat_v.at[pl.ds(0, b_w)]
        pltpu.sync_copy(pos_h.at[pl.ds(abase, b_w)], idxw[0])
        pltpu.sync_copy(cat_h.at[idxw[0]], catw)
        pltpu.make_async_copy(catw, ocat_h.at[pl.ds(abase, b_w)],
                              xsem[0]).start()
        it16 = lax.iota(jnp.int32, 16)

        @pl.loop(0, ii_w // 16)
        def _ext(t):
            jj = t * 16 + it16
            row = jj // topk
            lane = jj % topk
            nbr_v[pl.ds(t * 16, 16)] = plsc.load_gather(cat_v, [row, lane])

        pltpu.make_async_copy(catw, ocat_h.at[pl.ds(abase, b_w)],
                              xsem[0]).wait()

        def d_start(s, c, wait_reuse):
            if wait_reuse:
                pltpu.make_async_copy(rowsc[s], oii_h.at[pl.ds(0, ch)],
                                      wsem[s]).wait()
            pltpu.make_async_copy(emb_h.at[nbr_v.at[pl.ds(c * ch, ch)]],
                                  rowsc[s], gsem[s]).start()

        def d_finish(s, c):
            ibase = wid * ii_w + c * ch
            pltpu.make_async_copy(emb_h.at[nbr_v.at[pl.ds(c * ch, ch)]],
                                  rowsc[s], gsem[s]).wait()
            pltpu.make_async_copy(rowsc[s], oii_h.at[pl.ds(ibase, ch)],
                                  wsem[s]).start()

        run_job(ii_w // ch, d_start, d_finish)
        drain([(wsem[0], rowsc[0], oii_h.at[pl.ds(0, ch)]),
               (wsem[1], rowsc[1], oii_h.at[pl.ds(0, ch)])])

    return gather_kernel(emb_table, beta_uD, beta_iD, cat_pad,
                         src, pos, neg_t)


def _loss_block(src_ref, pos_ref, neg_ref, ii_ref, bu_ref, bip_ref,
                bin_ref, ocat_ref, bdn_ref, bdi_ref, one_ref, out_ref,
                *, total_b):
    b = pl.program_id(0)
    bb = src_ref.shape[1]
    d = src_ref.shape[2]
    nneg = neg_ref.shape[2] // d
    topk = ii_ref.shape[2] // d

    src = src_ref[0]
    pos = pos_ref[0]
    neg = neg_ref[0]
    ii = ii_ref[0]
    bdn = bdn_ref[...]
    bdi = bdi_ref[...]
    one = one_ref[...]

    src_n = jnp.concatenate([src] * nneg, axis=1)
    src_k = jnp.concatenate([src] * topk, axis=1)
    p_neg = src_n * neg
    p_ii = src_k * ii
    p_pos = src * pos
    neg_score = jnp.dot(p_neg, bdn)
    ii_score = jnp.dot(p_ii, bdi)
    pos_score = jnp.dot(p_pos, one)

    sq_neg = neg * neg
    sq_ii = ii * ii
    sq_sp = src * src + pos * pos
    l2n = jnp.dot(sq_neg, bdn)
    l2i = jnp.dot(sq_ii, bdi)
    l2sp = jnp.dot(sq_sp, one)

    bu = bu_ref[pl.ds(b, 1), :].reshape(bb, 1)
    bip = bip_ref[pl.ds(b, 1), :].reshape(bb, 1)
    bin_ = bin_ref[0]
    iisc = jax.lax.bitcast_convert_type(
        ocat_ref[0][:, topk:2 * topk], jnp.float32)

    def bce(x, target):
        return (jnp.maximum(x, 0.0) - x * target
                + jnp.log1p(jnp.exp(-jnp.abs(x))))

    pos_coe = 1.0 + _LAM * bu * bip
    neg_coe = 1.0 + _LAM * bu * bin_
    s_pos = jnp.sum(bce(pos_score, 1.0) * pos_coe)
    s_neg = jnp.sum(bce(neg_score, 0.0) * neg_coe)

    log_sig = jnp.minimum(ii_score, 0.0) - jnp.log1p(jnp.exp(-jnp.abs(ii_score)))
    s_i = jnp.sum(iisc * log_sig)

    s_l2 = jnp.sum(l2n) + jnp.sum(l2i) + jnp.sum(l2sp)

    n_pos = total_b
    n_neg = nneg * n_pos
    contrib = (s_pos / n_pos
               + (_NEG_WEIGHT / n_neg) * s_neg
               - _GAMMA * s_i
               + (0.5 * _L2_REG_WEIGHT) * s_l2)

    @pl.when(b == 0)
    def _():
        out_ref[0, 0] = 0.0
    out_ref[0, 0] += contrib


def _tc_loss(src_rows, pos_rows, neg_rows, ii_rows, ocat, bu, bip, bin_,
             topk, total_b):
    b, d = src_rows.shape
    nneg = neg_rows.shape[0] // b
    bb = 128
    nblk = b // bb

    src3 = src_rows.reshape(nblk, bb, d)
    pos3 = pos_rows.reshape(nblk, bb, d)
    neg4 = neg_rows.reshape(nblk, bb, nneg * d)
    ii4 = ii_rows.reshape(nblk, bb, topk * d)
    ocat3 = ocat.reshape(nblk, bb, d)
    bu2 = bu.reshape(nblk, bb)
    bip2 = bip.reshape(nblk, bb)
    bin2 = bin_.reshape(nblk, bb, nneg)

    def _bd(n):
        return (jax.lax.broadcasted_iota(jnp.int32, (n * d, n), 0) // d
                == jax.lax.broadcasted_iota(jnp.int32, (n * d, n), 1)
                ).astype(jnp.float32)

    bdn = _bd(nneg)
    bdi = _bd(topk)
    one = jnp.ones((d, 1), jnp.float32)

    out = pl.pallas_call(
        functools.partial(_loss_block, total_b=total_b),
        grid=(nblk,),
        in_specs=[
            pl.BlockSpec((1, bb, d), lambda i: (i, 0, 0)),
            pl.BlockSpec((1, bb, d), lambda i: (i, 0, 0)),
            pl.BlockSpec((1, bb, nneg * d), lambda i: (i, 0, 0)),
            pl.BlockSpec((1, bb, topk * d), lambda i: (i, 0, 0)),
            pl.BlockSpec((nblk, bb), lambda i: (0, 0)),
            pl.BlockSpec((nblk, bb), lambda i: (0, 0)),
            pl.BlockSpec((1, bb, nneg), lambda i: (i, 0, 0)),
            pl.BlockSpec((1, bb, d), lambda i: (i, 0, 0)),
            pl.BlockSpec((nneg * d, nneg), lambda i: (0, 0)),
            pl.BlockSpec((topk * d, topk), lambda i: (0, 0)),
            pl.BlockSpec((d, 1), lambda i: (0, 0)),
        ],
        out_specs=pl.BlockSpec((1, 1), lambda i: (0, 0),
                               memory_space=pltpu.SMEM),
        out_shape=jax.ShapeDtypeStruct((1, 1), jnp.float32),
    )(src3, pos3, neg4, ii4, bu2, bip2, bin2, ocat3, bdn, bdi, one)
    return out[0, 0]


def kernel(emb_table, beta_uD, beta_iD, ii_topk_similarity_scores,
           src, pos, neg, ii_topk_neighbors):
    b, nneg = neg.shape
    topk = ii_topk_neighbors.shape[1]

    d = emb_table.shape[1]

    neg_t = neg.reshape(-1).astype(jnp.int32)

    cat_pad = jnp.pad(
        jnp.concatenate(
            [ii_topk_neighbors.astype(jnp.int32),
             jax.lax.bitcast_convert_type(ii_topk_similarity_scores,
                                          jnp.int32)], axis=1),
        ((0, 0), (0, d - 2 * topk)))

    nslice = 4
    bs = b // nslice
    src_i = src.astype(jnp.int32)
    pos_i = pos.astype(jnp.int32)
    loss = jnp.float32(0.0)
    for si in range(nslice):
        sl = slice(si * bs, (si + 1) * bs)
        outs = _sc_gather(
            emb_table, beta_uD, beta_iD, cat_pad,
            src_i[sl], pos_i[sl], neg_t[si * bs * nneg:(si + 1) * bs * nneg],
            topk)
        loss = loss + _tc_loss(*outs, topk, b)
    return loss
